# baseline (speedup 1.0000x reference)
import functools

import jax
import jax.numpy as jnp
from jax import lax
from jax.experimental import pallas as pl
from jax.experimental.pallas import tpu as pltpu
from jax.experimental.pallas import tpu_sc as plsc

_LANES = 16


def _wid(info):
    return lax.axis_index("s") * info.num_cores + lax.axis_index("c")


def _make_tc_init(n_total, d, n_init, v_pad):
    blk = 512
    nblocks = n_init // blk
    dw = d // 2

    def body(ids_ref, tab_ref, store_ref):
        idv = ids_ref[0]
        iota = lax.broadcasted_iota(jnp.int32, (v_pad, blk), 0)
        m = ((iota == idv[0:1, :]) | (iota == idv[1:2, :])).astype(
            jnp.bfloat16)
        r = jax.lax.dot_general(m, tab_ref[...], (((0,), (0,)), ((), ())),
                                preferred_element_type=jnp.float32)
        store_ref[...] = pltpu.pack_elementwise(
            [r[:, :dw], r[:, dw:]], packed_dtype=jnp.bfloat16)

    return pl.pallas_call(
        body,
        grid=(nblocks,),
        in_specs=[
            pl.BlockSpec((1, 2, blk), lambda i: (i, 0, 0)),
            pl.BlockSpec((v_pad, d), lambda i: (0, 0)),
        ],
        out_specs=pl.BlockSpec((blk, dw), lambda i: (i, 0)),
        out_shape=jax.ShapeDtypeStruct((n_total, dw), jnp.int32),
    )


def _make_sc_gather(n_total, d, n_idx, n_layers, layer, info):
    nw = info.num_cores * info.num_subcores
    per = n_idx // nw
    chunk = 128
    nchunks = per // chunk
    half = nw // 2
    mesh = plsc.VectorSubcoreMesh(core_axis_name="c", subcore_axis_name="s")

    dw = d // 2

    @functools.partial(
        pl.kernel,
        out_type=jax.ShapeDtypeStruct((n_idx, dw), jnp.int32),
        mesh=mesh,
        scratch_types=[
            pltpu.VMEM((nchunks, chunk), jnp.int32),
            pltpu.VMEM((per, dw), jnp.int32),
            pltpu.SemaphoreType.DMA,
            pltpu.SemaphoreType.DMA,
        ],
    )
    def gather_k(store_hbm, idx_hbm, out_hbm, idx_v, rows_v, semg, semw):
        w = _wid(info)
        pltpu.sync_copy(idx_hbm.at[layer, w], idx_v)
        gathers = [
            pltpu.async_copy(store_hbm.at[idx_v.at[j]],
                             rows_v.at[pl.ds(j * chunk, chunk)], semg)
            for j in range(nchunks)
        ]
        writes = []
        for j in range(nchunks):
            gathers[j].wait()
            writes.append(pltpu.async_copy(
                rows_v.at[pl.ds(j * chunk, chunk)],
                out_hbm.at[pl.ds(w * per + j * chunk, chunk)], semw))
        for c in writes:
            c.wait()

    return gather_k


def _softplus_terms(m):
    t = jnp.log1p(jnp.exp(-jnp.abs(m)))
    sp_pos = jnp.maximum(m, 0.0) + t
    sp_neg = jnp.maximum(-m, 0.0) + t
    return sp_pos, sp_neg


def _eval_accumulate(e, ev1, ev2row, evb1, evb2_s, pos_row, neg_row,
                     acc_ref, d):
    n = e.shape[0]
    hh = jax.lax.dot(e, ev1, preferred_element_type=jnp.float32)
    hh = jnp.maximum(hh + evb1.reshape(1, d), 0.0)
    m = jax.lax.dot_general(ev2row, hh, (((1,), (1,)), ((), ())),
                            preferred_element_type=jnp.float32)
    m = m + evb2_s
    sp_pos, sp_neg = _softplus_terms(m)
    is_pos = (m >= 0.0).astype(jnp.float32)
    pos = pos_row.reshape(1, n)
    neg = neg_row.reshape(1, n)
    acc_ref[0:1, :] += pos * sp_neg
    acc_ref[1:2, :] += neg * sp_pos
    acc_ref[2:3, :] += pos
    acc_ref[3:4, :] += neg
    acc_ref[4:5, :] += pos * is_pos
    acc_ref[5:6, :] += neg * (1.0 - is_pos)


def _unpack_parities(x_i32, dtype=jnp.float32):
    lo = pltpu.unpack_elementwise(x_i32, index=0, packed_dtype=jnp.bfloat16,
                                  unpacked_dtype=dtype)
    hi = pltpu.unpack_elementwise(x_i32, index=1, packed_dtype=jnp.bfloat16,
                                  unpacked_dtype=dtype)
    return lo, hi


def _make_tc_mlp(n_total, d, npl, r_rules, base_row):
    npr = npl // r_rules
    dw = d // 2
    base_block = base_row // npr
    nb = base_row // npr

    def body(pa_ref, pb_ref, w1_ref, b1_ref, w2_ref, b2_ref,
             ev1_ref, ev2p_ref, evb1_ref, evb2_ref, pos_ref, neg_ref,
             store_ref, acc_in_ref, out_ref, acc_out_ref, accv_ref):
        r = pl.program_id(0)

        @pl.when(r == 0)
        def _():
            accv_ref[...] = jnp.zeros((8, 512), jnp.float32)

        w1 = w1_ref[0]
        dot = functools.partial(jax.lax.dot,
                                preferred_element_type=jnp.float32)
        pa_lo, pa_hi = _unpack_parities(pa_ref[...])
        pb_lo, pb_hi = _unpack_parities(pb_ref[...])
        h = dot(pa_lo.astype(jnp.bfloat16), w1[:dw])
        h = h + dot(pa_hi.astype(jnp.bfloat16), w1[dw:d])
        h = h + dot(pb_lo.astype(jnp.bfloat16), w1[d:d + dw])
        h = h + dot(pb_hi.astype(jnp.bfloat16), w1[d + dw:])
        h = jnp.maximum(h + b1_ref[0], 0.0)
        e = dot(h.astype(jnp.bfloat16), w2_ref[0])
        e = jnp.maximum(e + b2_ref[0], 0.0)
        out_ref[...] = pltpu.pack_elementwise(
            [e[:, :dw], e[:, dw:]], packed_dtype=jnp.bfloat16)
        _eval_accumulate(e, ev1_ref[...], ev2p_ref[...], evb1_ref[...],
                         evb2_ref[0], pos_ref[...], neg_ref[...], accv_ref, d)

        @pl.when(r == r_rules - 1)
        def _():
            acc_out_ref[...] = acc_in_ref[...] + accv_ref[...]

    return pl.pallas_call(
        body,
        grid=(r_rules,),
        in_specs=[
            pl.BlockSpec((npr, dw), lambda r: (r, 0)),
            pl.BlockSpec((npr, dw), lambda r: (r_rules + r, 0)),
            pl.BlockSpec((1, 2 * d, d), lambda r: (r, 0, 0)),
            pl.BlockSpec((1, 1, d), lambda r: (r, 0, 0)),
            pl.BlockSpec((1, d, d), lambda r: (r, 0, 0)),
            pl.BlockSpec((1, 1, d), lambda r: (r, 0, 0)),
            pl.BlockSpec((d, d), lambda r: (0, 0)),
            pl.BlockSpec((1, d), lambda r: (0, 0)),
            pl.BlockSpec((d,), lambda r: (0,)),
            pl.BlockSpec(memory_space=pltpu.MemorySpace.SMEM),
            pl.BlockSpec((1, 1, npr), lambda r: (nb + r, 0, 0)),
            pl.BlockSpec((1, 1, npr), lambda r: (nb + r, 0, 0)),
            pl.BlockSpec(memory_space=pltpu.MemorySpace.HBM),
            pl.BlockSpec((8, 512), lambda r: (0, 0)),
        ],
        out_specs=[
            pl.BlockSpec((npr, dw), lambda r: (base_block + r, 0)),
            pl.BlockSpec((8, 512), lambda r: (0, 0)),
        ],
        out_shape=[
            jax.ShapeDtypeStruct((n_total, dw), jnp.int32),
            jax.ShapeDtypeStruct((8, 512), jnp.float32),
        ],
        scratch_shapes=[pltpu.VMEM((8, 512), jnp.float32)],
        input_output_aliases={12: 0, 13: 1},
    )


def _make_tc_final(n_total, d, n_init, blk):
    nblocks = n_init // blk
    dw = d // 2

    def body(store_ref, ev1_ref, ev2p_ref, evb1_ref, evb2_ref,
             pos_ref, neg_ref, acc_in_ref,
             loss_ref, pok_ref, nok_ref, accv_ref):
        i = pl.program_id(0)

        @pl.when(i == 0)
        def _():
            accv_ref[...] = jnp.zeros((8, 512), jnp.float32)

        s_e, s_o = _unpack_parities(store_ref[...])
        _eval_accumulate(jnp.concatenate([s_e, s_o], axis=1),
                         ev1_ref[...], ev2p_ref[...], evb1_ref[...],
                         evb2_ref[0], pos_ref[...], neg_ref[...], accv_ref, d)

        @pl.when(i == nblocks - 1)
        def _():
            s = acc_in_ref[...] + accv_ref[...]
            a = jnp.sum(s[0, :])
            b = jnp.sum(s[1, :])
            tot_pos = jnp.sum(s[2, :])
            tot_neg = jnp.sum(s[3, :])
            loss_ref[...] = ((tot_neg / tot_pos) * a + b).reshape(1, 1)
            pok_ref[...] = jnp.sum(s[4, :]).reshape(1, 1)
            nok_ref[...] = jnp.sum(s[5, :]).reshape(1, 1)

    return pl.pallas_call(
        body,
        grid=(nblocks,),
        in_specs=[
            pl.BlockSpec((blk, dw), lambda i: (i, 0)),
            pl.BlockSpec((d, d), lambda i: (0, 0)),
            pl.BlockSpec((1, d), lambda i: (0, 0)),
            pl.BlockSpec((d,), lambda i: (0,)),
            pl.BlockSpec(memory_space=pltpu.MemorySpace.SMEM),
            pl.BlockSpec((1, 1, blk), lambda i: (i, 0, 0)),
            pl.BlockSpec((1, 1, blk), lambda i: (i, 0, 0)),
            pl.BlockSpec((8, 512), lambda i: (0, 0)),
        ],
        out_specs=[
            pl.BlockSpec((1, 1), lambda i: (0, 0)),
            pl.BlockSpec((1, 1), lambda i: (0, 0)),
            pl.BlockSpec((1, 1), lambda i: (0, 0)),
        ],
        out_shape=[
            jax.ShapeDtypeStruct((1, 1), jnp.float32),
            jax.ShapeDtypeStruct((1, 1), jnp.float32),
            jax.ShapeDtypeStruct((1, 1), jnp.float32),
        ],
        scratch_shapes=[pltpu.VMEM((8, 512), jnp.float32)],
    )


def kernel(thax_ids, sine_ids, pars, pos_vals, neg_vals, thax_table,
           sine_table, W1, b1, W2, b2, Ev1, evb1, Ev2, evb2):
    n_init = thax_ids.shape[0]
    n_layers, npl = pars.shape[0], pars.shape[1]
    d = thax_table.shape[1]
    r_rules = W1.shape[0]
    n_total = pos_vals.shape[0]
    info = plsc.get_sparse_core_info()
    nw = info.num_cores * info.num_subcores

    n_thax = thax_table.shape[0]
    v_pad = ((n_thax + sine_table.shape[0] + 7) // 8) * 8
    tab = jnp.concatenate([thax_table, sine_table], axis=0)
    tab = jnp.pad(tab, ((0, v_pad - tab.shape[0]), (0, 0)))
    tabp = tab.astype(jnp.bfloat16)
    ids2 = jnp.stack([thax_ids.astype(jnp.int32),
                      sine_ids.astype(jnp.int32) + n_thax],
                     axis=0).reshape(2, -1, 512).transpose(1, 0, 2)
    init_k = _make_tc_init(n_total, d, n_init, v_pad)
    store = init_k(ids2, tabp)

    idx_all = pars.astype(jnp.int32).transpose(0, 2, 1).reshape(
        n_layers, nw, -1, 128)
    w1b = W1.astype(jnp.bfloat16)
    w2b = W2.astype(jnp.bfloat16)
    b1r = b1.reshape(r_rules, 1, d)
    b2r = b2.reshape(r_rules, 1, d)
    ev2p = Ev2.reshape(1, d)
    pos3 = pos_vals.reshape(-1, 1, 512)
    neg3 = neg_vals.reshape(-1, 1, 512)
    acc = jnp.zeros((8, 512), jnp.float32)
    for l in range(n_layers):
        gather_k = _make_sc_gather(n_total, d, 2 * npl, n_layers, l, info)
        p = gather_k(store, idx_all)
        mlp_k = _make_tc_mlp(n_total, d, npl, r_rules, n_init + l * npl)
        store, acc = mlp_k(p, p, w1b, b1r, w2b, b2r, Ev1, ev2p, evb1, evb2,
                           pos3, neg3, store, acc)

    final_k = _make_tc_final(n_total, d, n_init, 512)
    loss2, pok2, nok2 = final_k(store, Ev1, ev2p, evb1, evb2,
                                pos3, neg3, acc)
    return loss2.reshape(1), pok2[0, 0], nok2[0, 0]


# speedup vs baseline: 1.0176x; 1.0176x over previous
import functools

import jax
import jax.numpy as jnp
from jax import lax
from jax.experimental import pallas as pl
from jax.experimental.pallas import tpu as pltpu
from jax.experimental.pallas import tpu_sc as plsc

_LANES = 16


def _wid(info):
    return lax.axis_index("s") * info.num_cores + lax.axis_index("c")


def _make_tc_init(n_total, d, n_init, v_pad):
    blk = 512
    nblocks = n_init // blk
    dw = d // 2

    def body(ids_ref, tab_ref, store_ref):
        idv = ids_ref[0]
        iota = lax.broadcasted_iota(jnp.int32, (v_pad, blk), 0)
        m = ((iota == idv[0:1, :]) | (iota == idv[1:2, :])).astype(
            jnp.bfloat16)
        r = jax.lax.dot_general(m, tab_ref[...], (((0,), (0,)), ((), ())),
                                preferred_element_type=jnp.float32)
        store_ref[...] = pltpu.pack_elementwise(
            [r[:, :dw], r[:, dw:]], packed_dtype=jnp.bfloat16)

    return pl.pallas_call(
        body,
        grid=(nblocks,),
        in_specs=[
            pl.BlockSpec((1, 2, blk), lambda i: (i, 0, 0)),
            pl.BlockSpec((v_pad, d), lambda i: (0, 0)),
        ],
        out_specs=pl.BlockSpec((blk, dw), lambda i: (i, 0)),
        out_shape=jax.ShapeDtypeStruct((n_total, dw), jnp.int32),
    )


def _make_sc_gather(n_total, d, n_idx, n_layers, layer, info):
    nw = info.num_cores * info.num_subcores
    per = n_idx // nw
    chunk = 128
    nchunks = per // chunk
    half = nw // 2
    mesh = plsc.VectorSubcoreMesh(core_axis_name="c", subcore_axis_name="s")

    dw = d // 2

    @functools.partial(
        pl.kernel,
        out_type=jax.ShapeDtypeStruct((n_idx, dw), jnp.int32),
        mesh=mesh,
        scratch_types=[
            pltpu.VMEM((nchunks, chunk), jnp.int32),
            pltpu.VMEM((per, dw), jnp.int32),
            pltpu.SemaphoreType.DMA,
            pltpu.SemaphoreType.DMA,
        ],
    )
    def gather_k(store_hbm, idx_hbm, out_hbm, idx_v, rows_v, semg, semw):
        w = _wid(info)
        pltpu.sync_copy(idx_hbm.at[layer, w], idx_v)
        gathers = [
            pltpu.async_copy(store_hbm.at[idx_v.at[j]],
                             rows_v.at[pl.ds(j * chunk, chunk)], semg)
            for j in range(nchunks)
        ]
        writes = []
        for j in range(nchunks):
            gathers[j].wait()
            writes.append(pltpu.async_copy(
                rows_v.at[pl.ds(j * chunk, chunk)],
                out_hbm.at[pl.ds(w * per + j * chunk, chunk)], semw))
        for c in writes:
            c.wait()

    return gather_k


def _softplus_terms(m):
    t = jnp.log1p(jnp.exp(-jnp.abs(m)))
    sp_pos = jnp.maximum(m, 0.0) + t
    sp_neg = jnp.maximum(-m, 0.0) + t
    return sp_pos, sp_neg


def _eval_accumulate(e, ev1, ev2row, evb1, evb2_s, pos_row, neg_row,
                     acc_ref, d):
    n = e.shape[0]
    hh = jax.lax.dot(e, ev1, preferred_element_type=jnp.float32)
    hh = jnp.maximum(hh + evb1.reshape(1, d), 0.0)
    m = jax.lax.dot_general(ev2row, hh, (((1,), (1,)), ((), ())),
                            preferred_element_type=jnp.float32)
    m = m + evb2_s
    sp_pos, sp_neg = _softplus_terms(m)
    is_pos = (m >= 0.0).astype(jnp.float32)
    pos = pos_row.reshape(1, n)
    neg = neg_row.reshape(1, n)
    acc_ref[0:1, :] += pos * sp_neg
    acc_ref[1:2, :] += neg * sp_pos
    acc_ref[2:3, :] += pos
    acc_ref[3:4, :] += neg
    acc_ref[4:5, :] += pos * is_pos
    acc_ref[5:6, :] += neg * (1.0 - is_pos)


def _unpack_parities(x_i32, dtype=jnp.float32):
    lo = pltpu.unpack_elementwise(x_i32, index=0, packed_dtype=jnp.bfloat16,
                                  unpacked_dtype=dtype)
    hi = pltpu.unpack_elementwise(x_i32, index=1, packed_dtype=jnp.bfloat16,
                                  unpacked_dtype=dtype)
    return lo, hi


def _make_tc_mlp(n_total, d, npl, r_rules, base_row):
    npr = npl // r_rules
    dw = d // 2
    nsteps = r_rules // 2
    nb2 = base_row // (2 * npr)

    def _rule_mlp(p_lo, p_hi, q_lo, q_hi, w1, b1, w2, b2):
        dot = functools.partial(jax.lax.dot,
                                preferred_element_type=jnp.float32)
        h = dot(p_lo.astype(jnp.bfloat16), w1[:dw])
        h = h + dot(p_hi.astype(jnp.bfloat16), w1[dw:d])
        h = h + dot(q_lo.astype(jnp.bfloat16), w1[d:d + dw])
        h = h + dot(q_hi.astype(jnp.bfloat16), w1[d + dw:])
        h = jnp.maximum(h + b1, 0.0)
        e = dot(h.astype(jnp.bfloat16), w2)
        return jnp.maximum(e + b2, 0.0)

    def body(pa_ref, pb_ref, w1_ref, b1_ref, w2_ref, b2_ref,
             ev1_ref, ev2p_ref, evb1_ref, evb2_ref, pos_ref, neg_ref,
             store_ref, acc_in_ref, out_ref, acc_out_ref, accv_ref):
        s = pl.program_id(0)

        @pl.when(s == 0)
        def _():
            accv_ref[...] = jnp.zeros((8, 512), jnp.float32)

        pa_lo, pa_hi = _unpack_parities(pa_ref[...])
        pb_lo, pb_hi = _unpack_parities(pb_ref[...])
        for u in range(2):
            rows = slice(u * npr, (u + 1) * npr)
            e = _rule_mlp(pa_lo[rows], pa_hi[rows], pb_lo[rows], pb_hi[rows],
                          w1_ref[u], b1_ref[u], w2_ref[u], b2_ref[u])
            out_ref[rows, :] = pltpu.pack_elementwise(
                [e[:, :dw], e[:, dw:]], packed_dtype=jnp.bfloat16)
            _eval_accumulate(e, ev1_ref[...], ev2p_ref[...], evb1_ref[...],
                             evb2_ref[0], pos_ref[u], neg_ref[u],
                             accv_ref, d)

        @pl.when(s == nsteps - 1)
        def _():
            acc_out_ref[...] = acc_in_ref[...] + accv_ref[...]

    return pl.pallas_call(
        body,
        grid=(nsteps,),
        in_specs=[
            pl.BlockSpec((2 * npr, dw), lambda s: (s, 0)),
            pl.BlockSpec((2 * npr, dw), lambda s: (nsteps + s, 0)),
            pl.BlockSpec((2, 2 * d, d), lambda s: (s, 0, 0)),
            pl.BlockSpec((2, 1, d), lambda s: (s, 0, 0)),
            pl.BlockSpec((2, d, d), lambda s: (s, 0, 0)),
            pl.BlockSpec((2, 1, d), lambda s: (s, 0, 0)),
            pl.BlockSpec((d, d), lambda s: (0, 0)),
            pl.BlockSpec((1, d), lambda s: (0, 0)),
            pl.BlockSpec((d,), lambda s: (0,)),
            pl.BlockSpec(memory_space=pltpu.MemorySpace.SMEM),
            pl.BlockSpec((2, 1, npr), lambda s: (nb2 + s, 0, 0)),
            pl.BlockSpec((2, 1, npr), lambda s: (nb2 + s, 0, 0)),
            pl.BlockSpec(memory_space=pltpu.MemorySpace.HBM),
            pl.BlockSpec((8, 512), lambda s: (0, 0)),
        ],
        out_specs=[
            pl.BlockSpec((2 * npr, dw), lambda s: (nb2 + s, 0)),
            pl.BlockSpec((8, 512), lambda s: (0, 0)),
        ],
        out_shape=[
            jax.ShapeDtypeStruct((n_total, dw), jnp.int32),
            jax.ShapeDtypeStruct((8, 512), jnp.float32),
        ],
        scratch_shapes=[pltpu.VMEM((8, 512), jnp.float32)],
        input_output_aliases={12: 0, 13: 1},
    )


def _make_tc_final(n_total, d, n_init, blk):
    nblocks = n_init // blk
    dw = d // 2

    def body(store_ref, ev1_ref, ev2p_ref, evb1_ref, evb2_ref,
             pos_ref, neg_ref, acc_in_ref,
             loss_ref, pok_ref, nok_ref, accv_ref):
        i = pl.program_id(0)

        @pl.when(i == 0)
        def _():
            accv_ref[...] = jnp.zeros((8, 512), jnp.float32)

        s_e, s_o = _unpack_parities(store_ref[...])
        _eval_accumulate(jnp.concatenate([s_e, s_o], axis=1),
                         ev1_ref[...], ev2p_ref[...], evb1_ref[...],
                         evb2_ref[0], pos_ref[...], neg_ref[...], accv_ref, d)

        @pl.when(i == nblocks - 1)
        def _():
            s = acc_in_ref[...] + accv_ref[...]
            a = jnp.sum(s[0, :])
            b = jnp.sum(s[1, :])
            tot_pos = jnp.sum(s[2, :])
            tot_neg = jnp.sum(s[3, :])
            loss_ref[...] = ((tot_neg / tot_pos) * a + b).reshape(1, 1)
            pok_ref[...] = jnp.sum(s[4, :]).reshape(1, 1)
            nok_ref[...] = jnp.sum(s[5, :]).reshape(1, 1)

    return pl.pallas_call(
        body,
        grid=(nblocks,),
        in_specs=[
            pl.BlockSpec((blk, dw), lambda i: (i, 0)),
            pl.BlockSpec((d, d), lambda i: (0, 0)),
            pl.BlockSpec((1, d), lambda i: (0, 0)),
            pl.BlockSpec((d,), lambda i: (0,)),
            pl.BlockSpec(memory_space=pltpu.MemorySpace.SMEM),
            pl.BlockSpec((1, 1, blk), lambda i: (i, 0, 0)),
            pl.BlockSpec((1, 1, blk), lambda i: (i, 0, 0)),
            pl.BlockSpec((8, 512), lambda i: (0, 0)),
        ],
        out_specs=[
            pl.BlockSpec((1, 1), lambda i: (0, 0)),
            pl.BlockSpec((1, 1), lambda i: (0, 0)),
            pl.BlockSpec((1, 1), lambda i: (0, 0)),
        ],
        out_shape=[
            jax.ShapeDtypeStruct((1, 1), jnp.float32),
            jax.ShapeDtypeStruct((1, 1), jnp.float32),
            jax.ShapeDtypeStruct((1, 1), jnp.float32),
        ],
        scratch_shapes=[pltpu.VMEM((8, 512), jnp.float32)],
    )


def kernel(thax_ids, sine_ids, pars, pos_vals, neg_vals, thax_table,
           sine_table, W1, b1, W2, b2, Ev1, evb1, Ev2, evb2):
    n_init = thax_ids.shape[0]
    n_layers, npl = pars.shape[0], pars.shape[1]
    d = thax_table.shape[1]
    r_rules = W1.shape[0]
    n_total = pos_vals.shape[0]
    info = plsc.get_sparse_core_info()
    nw = info.num_cores * info.num_subcores

    n_thax = thax_table.shape[0]
    v_pad = ((n_thax + sine_table.shape[0] + 7) // 8) * 8
    tab = jnp.concatenate([thax_table, sine_table], axis=0)
    tab = jnp.pad(tab, ((0, v_pad - tab.shape[0]), (0, 0)))
    tabp = tab.astype(jnp.bfloat16)
    ids2 = jnp.stack([thax_ids.astype(jnp.int32),
                      sine_ids.astype(jnp.int32) + n_thax],
                     axis=0).reshape(2, -1, 512).transpose(1, 0, 2)
    init_k = _make_tc_init(n_total, d, n_init, v_pad)
    store = init_k(ids2, tabp)

    idx_all = pars.astype(jnp.int32).transpose(0, 2, 1).reshape(
        n_layers, nw, -1, 128)
    w1b = W1.astype(jnp.bfloat16)
    w2b = W2.astype(jnp.bfloat16)
    b1r = b1.reshape(r_rules, 1, d)
    b2r = b2.reshape(r_rules, 1, d)
    ev2p = Ev2.reshape(1, d)
    pos3 = pos_vals.reshape(-1, 1, 512)
    neg3 = neg_vals.reshape(-1, 1, 512)
    acc = jnp.zeros((8, 512), jnp.float32)
    for l in range(n_layers):
        gather_k = _make_sc_gather(n_total, d, 2 * npl, n_layers, l, info)
        p = gather_k(store, idx_all)
        mlp_k = _make_tc_mlp(n_total, d, npl, r_rules, n_init + l * npl)
        store, acc = mlp_k(p, p, w1b, b1r, w2b, b2r, Ev1, ev2p, evb1, evb2,
                           pos3, neg3, store, acc)

    final_k = _make_tc_final(n_total, d, n_init, 512)
    loss2, pok2, nok2 = final_k(store, Ev1, ev2p, evb1, evb2,
                                pos3, neg3, acc)
    return loss2.reshape(1), pok2[0, 0], nok2[0, 0]
